# Initial kernel scaffold; baseline (speedup 1.0000x reference)
import jax
import jax.numpy as jnp
from jax.experimental import pallas as pl


def kernel(x, edge_index, W1, b1, W2, b2):
    raise NotImplementedError("write your pallas kernel here")


# speedup vs baseline: 9.6749x; 9.6749x over previous
import functools

import jax
import jax.numpy as jnp
from jax import lax
from jax.experimental import pallas as pl
from jax.experimental.pallas import tpu as pltpu
from jax.experimental.pallas import tpu_sc as plsc

N_NODES = 10000
N_EDGES = 320000
D_IN = 128
D_HID = 256

NPAD = 10240
DUMP = 10000
NC = 2
NS = 16
CH = 80
EPAD = NC * NS * CH * 128
EPW = EPAD // (NC * NS)
ROWS_PER_TILE = NPAD // NS


def _sc_counts_body(dst_hbm, out_hbm, dst_v, cnt_v):
    c = lax.axis_index("c")
    s = lax.axis_index("s")
    w = c * NS + s

    def zero_body(i, _):
        cnt_v[i, pl.ds(0, 16)] = jnp.zeros((16,), jnp.float32)
        cnt_v[i, pl.ds(16, 16)] = jnp.zeros((16,), jnp.float32)
        cnt_v[i, pl.ds(32, 16)] = jnp.zeros((16,), jnp.float32)
        cnt_v[i, pl.ds(48, 16)] = jnp.zeros((16,), jnp.float32)
        cnt_v[i, pl.ds(64, 16)] = jnp.zeros((16,), jnp.float32)
        cnt_v[i, pl.ds(80, 16)] = jnp.zeros((16,), jnp.float32)
        cnt_v[i, pl.ds(96, 16)] = jnp.zeros((16,), jnp.float32)
        cnt_v[i, pl.ds(112, 16)] = jnp.zeros((16,), jnp.float32)
        return 0

    lax.fori_loop(0, NPAD // 128, zero_body, 0)
    pltpu.sync_copy(dst_hbm.at[pl.ds(w * EPW, EPW)], dst_v)
    ones = jnp.ones((16,), jnp.float32)

    def body(i, _):
        idx = dst_v[pl.ds(i * 16, 16)]
        row = lax.shift_right_logical(idx, 7)
        col = jnp.bitwise_and(idx, 127)
        plsc.addupdate_scatter(cnt_v, [row, col], ones)
        return 0

    lax.fori_loop(0, EPW // 16, body, 0)
    pltpu.sync_copy(cnt_v, out_hbm.at[w])


def _sc_scatter_body(table_hbm, src_hbm, dst_hbm, out_hbm, src_v, dst_v, buf,
                     acc, sem):
    c = lax.axis_index("c")
    s = lax.axis_index("s")
    w = c * NS + s

    def zero_body(i, _):
        buf[i, pl.ds(0, 16)] = jnp.zeros((16,), jnp.float32)
        buf[i, pl.ds(16, 16)] = jnp.zeros((16,), jnp.float32)
        buf[i, pl.ds(32, 16)] = jnp.zeros((16,), jnp.float32)
        buf[i, pl.ds(48, 16)] = jnp.zeros((16,), jnp.float32)
        buf[i, pl.ds(64, 16)] = jnp.zeros((16,), jnp.float32)
        buf[i, pl.ds(80, 16)] = jnp.zeros((16,), jnp.float32)
        buf[i, pl.ds(96, 16)] = jnp.zeros((16,), jnp.float32)
        buf[i, pl.ds(112, 16)] = jnp.zeros((16,), jnp.float32)
        return 0

    lax.fori_loop(0, 128, zero_body, 0)
    for k in range(ROWS_PER_TILE // 128):
        pltpu.sync_copy(buf, acc.at[pl.ds(s * ROWS_PER_TILE + k * 128, 128)])

    pltpu.sync_copy(src_hbm.at[w], src_v)
    pltpu.sync_copy(dst_hbm.at[w], dst_v)
    plsc.subcore_barrier()

    def body(ch, _):
        pltpu.async_copy(table_hbm.at[src_v.at[ch]], buf, sem).wait()
        pltpu.sync_copy(buf, acc.at[dst_v.at[ch]], add=True)
        return 0

    lax.fori_loop(0, CH, body, 0)
    plsc.subcore_barrier()
    pltpu.sync_copy(
        acc.at[pl.ds(s * ROWS_PER_TILE, ROWS_PER_TILE)],
        out_hbm.at[pl.ds(c * NPAD + s * ROWS_PER_TILE, ROWS_PER_TILE)])


@functools.cache
def _sc_kernels():
    mesh = plsc.VectorSubcoreMesh(core_axis_name="c", subcore_axis_name="s")
    params = pltpu.CompilerParams(needs_layout_passes=False)
    counts = pl.kernel(
        _sc_counts_body,
        out_type=jax.ShapeDtypeStruct((NC * NS, NPAD // 128, 128),
                                      jnp.float32),
        mesh=mesh,
        scratch_types=[
            pltpu.VMEM((EPW,), jnp.int32),
            pltpu.VMEM((NPAD // 128, 128), jnp.float32),
        ],
        compiler_params=params,
    )
    scatter = pl.kernel(
        _sc_scatter_body,
        out_type=jax.ShapeDtypeStruct((NC * NPAD, D_IN), jnp.float32),
        mesh=mesh,
        scratch_types=[
            pltpu.VMEM((CH, 128), jnp.int32),
            pltpu.VMEM((CH, 128), jnp.int32),
            pltpu.VMEM((128, D_IN), jnp.float32),
            pltpu.VMEM_SHARED((NPAD, D_IN), jnp.float32),
            pltpu.SemaphoreType.DMA,
        ],
        compiler_params=params,
    )
    return counts, scatter


def _tc_dinv_body(cnt_ref, out_ref):
    deg = jnp.sum(cnt_ref[...], axis=0) + 1.0
    out_ref[...] = lax.rsqrt(deg)


def _tc_scale_body(x_ref, dinv_ref, out_ref):
    out_ref[...] = x_ref[...] * dinv_ref[...]


def _tc_mid_body(acc_ref, xs_ref, dinv_ref, w1_ref, b1_ref, w2_ref, out_ref):
    dinv = dinv_ref[...]
    z = (acc_ref[0] + acc_ref[1] + xs_ref[...]) * dinv
    h1 = jnp.dot(z, w1_ref[...], preferred_element_type=jnp.float32)
    h1 = jnp.maximum(h1 + b1_ref[...], 0.0)
    out_ref[...] = jnp.dot(h1, w2_ref[...],
                           preferred_element_type=jnp.float32) * dinv


def _tc_out_body(acc_ref, g2_ref, dinv_ref, b2_ref, out_ref):
    pre = (acc_ref[0] + acc_ref[1] + g2_ref[...]) * dinv_ref[...]
    out_ref[...] = jnp.maximum(pre + b2_ref[...], 0.0)


def kernel(x, edge_index, W1, b1, W2, b2):
    src = edge_index[0]
    dst = edge_index[1]
    pad = jnp.full((EPAD - N_EDGES,), DUMP, dtype=jnp.int32)
    src_p = jnp.concatenate([src, pad])
    dst_p = jnp.concatenate([dst, pad])
    src_r = src_p.reshape(NC * NS, CH, 128)
    dst_r = dst_p.reshape(NC * NS, CH, 128)

    _sc_counts, _sc_scatter = _sc_kernels()
    counts = _sc_counts(dst_p)

    dinv = pl.pallas_call(
        _tc_dinv_body,
        out_shape=jax.ShapeDtypeStruct((NPAD // 128, 128), jnp.float32),
    )(counts)
    dinv_col = dinv.reshape(NPAD, 1)

    xpad = jnp.pad(x, ((0, NPAD - N_NODES), (0, 0)))
    xs = pl.pallas_call(
        _tc_scale_body,
        grid=(5,),
        in_specs=[
            pl.BlockSpec((2048, D_IN), lambda i: (i, 0)),
            pl.BlockSpec((2048, 1), lambda i: (i, 0)),
        ],
        out_specs=pl.BlockSpec((2048, D_IN), lambda i: (i, 0)),
        out_shape=jax.ShapeDtypeStruct((NPAD, D_IN), jnp.float32),
    )(xpad, dinv_col)

    acc1 = _sc_scatter(xs, src_r, dst_r).reshape(NC, NPAD, D_IN)

    g2 = pl.pallas_call(
        _tc_mid_body,
        grid=(8,),
        in_specs=[
            pl.BlockSpec((NC, 1280, D_IN), lambda i: (0, i, 0)),
            pl.BlockSpec((1280, D_IN), lambda i: (i, 0)),
            pl.BlockSpec((1280, 1), lambda i: (i, 0)),
            pl.BlockSpec((D_IN, D_HID), lambda i: (0, 0)),
            pl.BlockSpec((1, D_HID), lambda i: (0, 0)),
            pl.BlockSpec((D_HID, D_IN), lambda i: (0, 0)),
        ],
        out_specs=pl.BlockSpec((1280, D_IN), lambda i: (i, 0)),
        out_shape=jax.ShapeDtypeStruct((NPAD, D_IN), jnp.float32),
    )(acc1, xs, dinv_col, W1, b1.reshape(1, D_HID), W2)

    acc2 = _sc_scatter(g2, src_r, dst_r).reshape(NC, NPAD, D_IN)

    out = pl.pallas_call(
        _tc_out_body,
        grid=(5,),
        in_specs=[
            pl.BlockSpec((NC, 2000, D_IN), lambda i: (0, i, 0)),
            pl.BlockSpec((2000, D_IN), lambda i: (i, 0)),
            pl.BlockSpec((2000, 1), lambda i: (i, 0)),
            pl.BlockSpec((1, D_IN), lambda i: (0, 0)),
        ],
        out_specs=pl.BlockSpec((2000, D_IN), lambda i: (i, 0)),
        out_shape=jax.ShapeDtypeStruct((N_NODES, D_IN), jnp.float32),
    )(acc2[:, :N_NODES], g2[:N_NODES], dinv_col[:N_NODES],
      b2.reshape(1, D_IN))

    return out


# speedup vs baseline: 32.1410x; 3.3221x over previous
import functools

import jax
import jax.numpy as jnp
from jax import lax
from jax.experimental import pallas as pl
from jax.experimental.pallas import tpu as pltpu
from jax.experimental.pallas import tpu_sc as plsc

N_NODES = 10000
N_EDGES = 320000
D_IN = 128
D_HID = 256

NPAD = 10240
DUMP = 10000
NC = 2
NS = 16
CH = 80
HCH = 40
EPAD = NC * NS * CH * 128
EPW = EPAD // (NC * NS)
ROWS_PER_TILE = NPAD // NS


def _sc_counts_body(dst_hbm, out_hbm, dst_v, cnt_v):
    c = lax.axis_index("c")
    s = lax.axis_index("s")
    w = c * NS + s

    def zero_body(i, _):
        cnt_v[i, pl.ds(0, 16)] = jnp.zeros((16,), jnp.float32)
        cnt_v[i, pl.ds(16, 16)] = jnp.zeros((16,), jnp.float32)
        cnt_v[i, pl.ds(32, 16)] = jnp.zeros((16,), jnp.float32)
        cnt_v[i, pl.ds(48, 16)] = jnp.zeros((16,), jnp.float32)
        cnt_v[i, pl.ds(64, 16)] = jnp.zeros((16,), jnp.float32)
        cnt_v[i, pl.ds(80, 16)] = jnp.zeros((16,), jnp.float32)
        cnt_v[i, pl.ds(96, 16)] = jnp.zeros((16,), jnp.float32)
        cnt_v[i, pl.ds(112, 16)] = jnp.zeros((16,), jnp.float32)
        return 0

    lax.fori_loop(0, NPAD // 128, zero_body, 0)
    pltpu.sync_copy(dst_hbm.at[pl.ds(w * EPW, EPW)], dst_v)
    ones = jnp.ones((16,), jnp.float32)

    def body(i, _):
        idx = dst_v[pl.ds(i * 16, 16)]
        row = lax.shift_right_logical(idx, 7)
        col = jnp.bitwise_and(idx, 127)
        plsc.addupdate_scatter(cnt_v, [row, col], ones)
        return 0

    lax.fori_loop(0, EPW // 16, body, 0)
    pltpu.sync_copy(cnt_v, out_hbm.at[w])


def _sc_scatter_body(table_hbm, src_hbm, dst_hbm, out_hbm, src_v, dst_v, buf,
                     buf2, acc, sem, sem2):
    c = lax.axis_index("c")
    s = lax.axis_index("s")
    w = c * NS + s

    def zero_body(i, _):
        buf[i, pl.ds(0, 16)] = jnp.zeros((16,), jnp.float32)
        buf[i, pl.ds(16, 16)] = jnp.zeros((16,), jnp.float32)
        buf[i, pl.ds(32, 16)] = jnp.zeros((16,), jnp.float32)
        buf[i, pl.ds(48, 16)] = jnp.zeros((16,), jnp.float32)
        buf[i, pl.ds(64, 16)] = jnp.zeros((16,), jnp.float32)
        buf[i, pl.ds(80, 16)] = jnp.zeros((16,), jnp.float32)
        buf[i, pl.ds(96, 16)] = jnp.zeros((16,), jnp.float32)
        buf[i, pl.ds(112, 16)] = jnp.zeros((16,), jnp.float32)
        return 0

    lax.fori_loop(0, 128, zero_body, 0)
    for k in range(ROWS_PER_TILE // 128):
        pltpu.sync_copy(buf, acc.at[pl.ds(s * ROWS_PER_TILE + k * 128, 128)])
    plsc.subcore_barrier()

    def do_half(h):
        base = w * CH + h * HCH
        pltpu.sync_copy(src_hbm.at[pl.ds(base, HCH)], src_v)
        pltpu.sync_copy(dst_hbm.at[pl.ds(base, HCH)], dst_v)
        pltpu.async_copy(table_hbm.at[src_v.at[0]], buf, sem)

        def body(g, _):
            e0 = 2 * g
            pltpu.make_async_copy(table_hbm.at[src_v.at[e0]], buf,
                                  sem).wait()
            pltpu.async_copy(table_hbm.at[src_v.at[e0 + 1]], buf2, sem2)
            pltpu.sync_copy(buf, acc.at[dst_v.at[e0]], add=True)
            pltpu.make_async_copy(table_hbm.at[src_v.at[e0 + 1]], buf2,
                                  sem2).wait()
            nxt = jnp.minimum(e0 + 2, HCH - 1)
            pltpu.async_copy(table_hbm.at[src_v.at[nxt]], buf, sem)
            pltpu.sync_copy(buf2, acc.at[dst_v.at[e0 + 1]], add=True)
            return 0

        lax.fori_loop(0, HCH // 2, body, 0)
        pltpu.make_async_copy(table_hbm.at[src_v.at[HCH - 1]], buf,
                              sem).wait()

    do_half(0)
    do_half(1)
    plsc.subcore_barrier()
    pltpu.sync_copy(
        acc.at[pl.ds(s * ROWS_PER_TILE, ROWS_PER_TILE)],
        out_hbm.at[pl.ds(c * NPAD + s * ROWS_PER_TILE, ROWS_PER_TILE)])


@functools.cache
def _sc_kernels():
    mesh = plsc.VectorSubcoreMesh(core_axis_name="c", subcore_axis_name="s")
    params = pltpu.CompilerParams(needs_layout_passes=False)
    counts = pl.kernel(
        _sc_counts_body,
        out_type=jax.ShapeDtypeStruct((NC * NS, NPAD // 128, 128),
                                      jnp.float32),
        mesh=mesh,
        scratch_types=[
            pltpu.VMEM((EPW,), jnp.int32),
            pltpu.VMEM((NPAD // 128, 128), jnp.float32),
        ],
        compiler_params=params,
    )
    scatter = pl.kernel(
        _sc_scatter_body,
        out_type=jax.ShapeDtypeStruct((NC * NPAD, D_IN), jnp.float32),
        mesh=mesh,
        scratch_types=[
            pltpu.VMEM((HCH, 128), jnp.int32),
            pltpu.VMEM((HCH, 128), jnp.int32),
            pltpu.VMEM((128, D_IN), jnp.float32),
            pltpu.VMEM((128, D_IN), jnp.float32),
            pltpu.VMEM_SHARED((NPAD, D_IN), jnp.float32),
            pltpu.SemaphoreType.DMA,
            pltpu.SemaphoreType.DMA,
        ],
        compiler_params=params,
    )
    return counts, scatter


def _tc_dinv_body(cnt_ref, out_ref):
    deg = jnp.sum(cnt_ref[...], axis=0) + 1.0
    out_ref[...] = lax.rsqrt(deg)


def _tc_scale_body(x_ref, dinv_ref, out_ref):
    out_ref[...] = x_ref[...] * dinv_ref[...]


def _tc_mid_body(acc_ref, xs_ref, dinv_ref, w1_ref, b1_ref, w2_ref, out_ref):
    dinv = dinv_ref[...]
    z = (acc_ref[0] + acc_ref[1] + xs_ref[...]) * dinv
    h1 = jnp.dot(z, w1_ref[...], preferred_element_type=jnp.float32)
    h1 = jnp.maximum(h1 + b1_ref[...], 0.0)
    out_ref[...] = jnp.dot(h1, w2_ref[...],
                           preferred_element_type=jnp.float32) * dinv


def _tc_out_body(acc_ref, g2_ref, dinv_ref, b2_ref, out_ref):
    pre = (acc_ref[0] + acc_ref[1] + g2_ref[...]) * dinv_ref[...]
    out_ref[...] = jnp.maximum(pre + b2_ref[...], 0.0)


def kernel(x, edge_index, W1, b1, W2, b2):
    src = edge_index[0]
    dst = edge_index[1]
    pad = DUMP + jnp.arange(EPAD - N_EDGES, dtype=jnp.int32) % (NPAD - DUMP)
    src_p = jnp.concatenate([src, pad])
    dst_p = jnp.concatenate([dst, pad])
    src_r = src_p.reshape(NC * NS * CH, 128)
    dst_r = dst_p.reshape(NC * NS * CH, 128)

    _sc_counts, _sc_scatter = _sc_kernels()
    counts = _sc_counts(dst_p)

    dinv = pl.pallas_call(
        _tc_dinv_body,
        out_shape=jax.ShapeDtypeStruct((NPAD // 128, 128), jnp.float32),
    )(counts)
    dinv_col = dinv.reshape(NPAD, 1)

    xpad = jnp.pad(x, ((0, NPAD - N_NODES), (0, 0)))
    xs = pl.pallas_call(
        _tc_scale_body,
        grid=(5,),
        in_specs=[
            pl.BlockSpec((2048, D_IN), lambda i: (i, 0)),
            pl.BlockSpec((2048, 1), lambda i: (i, 0)),
        ],
        out_specs=pl.BlockSpec((2048, D_IN), lambda i: (i, 0)),
        out_shape=jax.ShapeDtypeStruct((NPAD, D_IN), jnp.float32),
    )(xpad, dinv_col)

    acc1 = _sc_scatter(xs, src_r, dst_r).reshape(NC, NPAD, D_IN)

    g2 = pl.pallas_call(
        _tc_mid_body,
        grid=(8,),
        in_specs=[
            pl.BlockSpec((NC, 1280, D_IN), lambda i: (0, i, 0)),
            pl.BlockSpec((1280, D_IN), lambda i: (i, 0)),
            pl.BlockSpec((1280, 1), lambda i: (i, 0)),
            pl.BlockSpec((D_IN, D_HID), lambda i: (0, 0)),
            pl.BlockSpec((1, D_HID), lambda i: (0, 0)),
            pl.BlockSpec((D_HID, D_IN), lambda i: (0, 0)),
        ],
        out_specs=pl.BlockSpec((1280, D_IN), lambda i: (i, 0)),
        out_shape=jax.ShapeDtypeStruct((NPAD, D_IN), jnp.float32),
    )(acc1, xs, dinv_col, W1, b1.reshape(1, D_HID), W2)

    acc2 = _sc_scatter(g2, src_r, dst_r).reshape(NC, NPAD, D_IN)

    out = pl.pallas_call(
        _tc_out_body,
        grid=(5,),
        in_specs=[
            pl.BlockSpec((NC, 2000, D_IN), lambda i: (0, i, 0)),
            pl.BlockSpec((2000, D_IN), lambda i: (i, 0)),
            pl.BlockSpec((2000, 1), lambda i: (i, 0)),
            pl.BlockSpec((1, D_IN), lambda i: (0, 0)),
        ],
        out_specs=pl.BlockSpec((2000, D_IN), lambda i: (i, 0)),
        out_shape=jax.ShapeDtypeStruct((N_NODES, D_IN), jnp.float32),
    )(acc2[:, :N_NODES], g2[:N_NODES], dinv_col[:N_NODES],
      b2.reshape(1, D_IN))

    return out


# speedup vs baseline: 37.3883x; 1.1633x over previous
import functools

import jax
import jax.numpy as jnp
from jax import lax
from jax.experimental import pallas as pl
from jax.experimental.pallas import tpu as pltpu
from jax.experimental.pallas import tpu_sc as plsc

N_NODES = 10000
N_EDGES = 320000
D_IN = 128
D_HID = 256

NPAD = 10240
DUMP = 10000
NC = 2
NS = 16
CH = 80
HCH = 40
EPAD = NC * NS * CH * 128
EPW = EPAD // (NC * NS)
ROWS_PER_TILE = NPAD // NS


def _sc_counts_body(dst_hbm, out_hbm, dst_v, cnt_v):
    c = lax.axis_index("c")
    s = lax.axis_index("s")
    w = c * NS + s

    def zero_body(i, _):
        cnt_v[i, pl.ds(0, 16)] = jnp.zeros((16,), jnp.float32)
        cnt_v[i, pl.ds(16, 16)] = jnp.zeros((16,), jnp.float32)
        cnt_v[i, pl.ds(32, 16)] = jnp.zeros((16,), jnp.float32)
        cnt_v[i, pl.ds(48, 16)] = jnp.zeros((16,), jnp.float32)
        cnt_v[i, pl.ds(64, 16)] = jnp.zeros((16,), jnp.float32)
        cnt_v[i, pl.ds(80, 16)] = jnp.zeros((16,), jnp.float32)
        cnt_v[i, pl.ds(96, 16)] = jnp.zeros((16,), jnp.float32)
        cnt_v[i, pl.ds(112, 16)] = jnp.zeros((16,), jnp.float32)
        return 0

    lax.fori_loop(0, NPAD // 128, zero_body, 0)
    pltpu.sync_copy(dst_hbm.at[pl.ds(w * EPW, EPW)], dst_v)
    ones = jnp.ones((16,), jnp.float32)

    def body(i, _):
        idx = dst_v[pl.ds(i * 16, 16)]
        row = lax.shift_right_logical(idx, 7)
        col = jnp.bitwise_and(idx, 127)
        plsc.addupdate_scatter(cnt_v, [row, col], ones)
        return 0

    lax.fori_loop(0, EPW // 16, body, 0)
    pltpu.sync_copy(cnt_v, out_hbm.at[w])


def _sc_scatter_body(table_hbm, src_hbm, dst_hbm, out_hbm, src_v, dst_v, buf,
                     buf2, acc, sem, sem2):
    c = lax.axis_index("c")
    s = lax.axis_index("s")
    w = c * NS + s

    def zero_body(i, _):
        buf[i, pl.ds(0, 16)] = jnp.zeros((16,), jnp.float32)
        buf[i, pl.ds(16, 16)] = jnp.zeros((16,), jnp.float32)
        buf[i, pl.ds(32, 16)] = jnp.zeros((16,), jnp.float32)
        buf[i, pl.ds(48, 16)] = jnp.zeros((16,), jnp.float32)
        buf[i, pl.ds(64, 16)] = jnp.zeros((16,), jnp.float32)
        buf[i, pl.ds(80, 16)] = jnp.zeros((16,), jnp.float32)
        buf[i, pl.ds(96, 16)] = jnp.zeros((16,), jnp.float32)
        buf[i, pl.ds(112, 16)] = jnp.zeros((16,), jnp.float32)
        return 0

    lax.fori_loop(0, 128, zero_body, 0)
    for k in range(ROWS_PER_TILE // 128):
        pltpu.sync_copy(buf, acc.at[pl.ds(s * ROWS_PER_TILE + k * 128, 128)])
    plsc.subcore_barrier()

    def do_half(h):
        base = w * CH + h * HCH
        pltpu.sync_copy(src_hbm.at[pl.ds(base, HCH)], src_v)
        pltpu.sync_copy(dst_hbm.at[pl.ds(base, HCH)], dst_v)
        pltpu.async_copy(table_hbm.at[src_v.at[0]], buf, sem)
        pltpu.async_copy(table_hbm.at[src_v.at[1]], buf2, sem2)

        def body(g, _):
            e0 = 2 * g
            pltpu.make_async_copy(table_hbm.at[src_v.at[e0]], buf,
                                  sem).wait()
            pltpu.sync_copy(buf, acc.at[dst_v.at[e0]], add=True)
            pltpu.async_copy(
                table_hbm.at[src_v.at[jnp.minimum(e0 + 2, HCH - 1)]], buf,
                sem)
            pltpu.make_async_copy(table_hbm.at[src_v.at[e0 + 1]], buf2,
                                  sem2).wait()
            pltpu.sync_copy(buf2, acc.at[dst_v.at[e0 + 1]], add=True)
            pltpu.async_copy(
                table_hbm.at[src_v.at[jnp.minimum(e0 + 3, HCH - 1)]], buf2,
                sem2)
            return 0

        lax.fori_loop(0, HCH // 2, body, 0)
        pltpu.make_async_copy(table_hbm.at[src_v.at[HCH - 1]], buf,
                              sem).wait()
        pltpu.make_async_copy(table_hbm.at[src_v.at[HCH - 1]], buf2,
                              sem2).wait()

    do_half(0)
    do_half(1)
    plsc.subcore_barrier()
    pltpu.sync_copy(
        acc.at[pl.ds(s * ROWS_PER_TILE, ROWS_PER_TILE)],
        out_hbm.at[pl.ds(c * NPAD + s * ROWS_PER_TILE, ROWS_PER_TILE)])


@functools.cache
def _sc_kernels():
    mesh = plsc.VectorSubcoreMesh(core_axis_name="c", subcore_axis_name="s")
    params = pltpu.CompilerParams(needs_layout_passes=False)
    counts = pl.kernel(
        _sc_counts_body,
        out_type=jax.ShapeDtypeStruct((NC * NS, NPAD // 128, 128),
                                      jnp.float32),
        mesh=mesh,
        scratch_types=[
            pltpu.VMEM((EPW,), jnp.int32),
            pltpu.VMEM((NPAD // 128, 128), jnp.float32),
        ],
        compiler_params=params,
    )
    scatter = pl.kernel(
        _sc_scatter_body,
        out_type=jax.ShapeDtypeStruct((NC * NPAD, D_IN), jnp.float32),
        mesh=mesh,
        scratch_types=[
            pltpu.VMEM((HCH, 128), jnp.int32),
            pltpu.VMEM((HCH, 128), jnp.int32),
            pltpu.VMEM((128, D_IN), jnp.float32),
            pltpu.VMEM((128, D_IN), jnp.float32),
            pltpu.VMEM_SHARED((NPAD, D_IN), jnp.float32),
            pltpu.SemaphoreType.DMA,
            pltpu.SemaphoreType.DMA,
        ],
        compiler_params=params,
    )
    return counts, scatter


def _tc_dinv_body(cnt_ref, out_ref):
    deg = jnp.sum(cnt_ref[...], axis=0) + 1.0
    out_ref[...] = lax.rsqrt(deg)


def _tc_scale_body(x_ref, dinv_ref, out_ref):
    out_ref[...] = x_ref[...] * dinv_ref[...]


def _tc_mid_body(acc_ref, xs_ref, dinv_ref, w1_ref, b1_ref, w2_ref, out_ref):
    dinv = dinv_ref[...]
    z = (acc_ref[0] + acc_ref[1] + xs_ref[...]) * dinv
    h1 = jnp.dot(z, w1_ref[...], preferred_element_type=jnp.float32)
    h1 = jnp.maximum(h1 + b1_ref[...], 0.0)
    out_ref[...] = jnp.dot(h1, w2_ref[...],
                           preferred_element_type=jnp.float32) * dinv


def _tc_out_body(acc_ref, g2_ref, dinv_ref, b2_ref, out_ref):
    pre = (acc_ref[0] + acc_ref[1] + g2_ref[...]) * dinv_ref[...]
    out_ref[...] = jnp.maximum(pre + b2_ref[...], 0.0)


def kernel(x, edge_index, W1, b1, W2, b2):
    src = edge_index[0]
    dst = edge_index[1]
    pad = DUMP + jnp.arange(EPAD - N_EDGES, dtype=jnp.int32) % (NPAD - DUMP)
    src_p = jnp.concatenate([src, pad])
    dst_p = jnp.concatenate([dst, pad])
    src_r = src_p.reshape(NC * NS * CH, 128)
    dst_r = dst_p.reshape(NC * NS * CH, 128)

    _sc_counts, _sc_scatter = _sc_kernels()
    counts = _sc_counts(dst_p)

    dinv = pl.pallas_call(
        _tc_dinv_body,
        out_shape=jax.ShapeDtypeStruct((NPAD // 128, 128), jnp.float32),
    )(counts)
    dinv_col = dinv.reshape(NPAD, 1)

    xs = pl.pallas_call(
        _tc_scale_body,
        grid=(5,),
        in_specs=[
            pl.BlockSpec((2048, D_IN), lambda i: (i, 0)),
            pl.BlockSpec((2048, 1), lambda i: (i, 0)),
        ],
        out_specs=pl.BlockSpec((2048, D_IN), lambda i: (i, 0)),
        out_shape=jax.ShapeDtypeStruct((NPAD, D_IN), jnp.float32),
    )(x, dinv_col)

    acc1 = _sc_scatter(xs, src_r, dst_r).reshape(NC, NPAD, D_IN)

    g2 = pl.pallas_call(
        _tc_mid_body,
        grid=(8,),
        in_specs=[
            pl.BlockSpec((NC, 1280, D_IN), lambda i: (0, i, 0)),
            pl.BlockSpec((1280, D_IN), lambda i: (i, 0)),
            pl.BlockSpec((1280, 1), lambda i: (i, 0)),
            pl.BlockSpec((D_IN, D_HID), lambda i: (0, 0)),
            pl.BlockSpec((1, D_HID), lambda i: (0, 0)),
            pl.BlockSpec((D_HID, D_IN), lambda i: (0, 0)),
        ],
        out_specs=pl.BlockSpec((1280, D_IN), lambda i: (i, 0)),
        out_shape=jax.ShapeDtypeStruct((NPAD, D_IN), jnp.float32),
    )(acc1, xs, dinv_col, W1, b1.reshape(1, D_HID), W2)

    acc2 = _sc_scatter(g2, src_r, dst_r).reshape(NC, NPAD, D_IN)

    out = pl.pallas_call(
        _tc_out_body,
        grid=(5,),
        in_specs=[
            pl.BlockSpec((NC, 2000, D_IN), lambda i: (0, i, 0)),
            pl.BlockSpec((2000, D_IN), lambda i: (i, 0)),
            pl.BlockSpec((2000, 1), lambda i: (i, 0)),
            pl.BlockSpec((1, D_IN), lambda i: (0, 0)),
        ],
        out_specs=pl.BlockSpec((2000, D_IN), lambda i: (i, 0)),
        out_shape=jax.ShapeDtypeStruct((N_NODES, D_IN), jnp.float32),
    )(acc2, g2, dinv_col, b2.reshape(1, D_IN))

    return out


# speedup vs baseline: 39.6272x; 1.0599x over previous
import functools

import jax
import jax.numpy as jnp
from jax import lax
from jax.experimental import pallas as pl
from jax.experimental.pallas import tpu as pltpu
from jax.experimental.pallas import tpu_sc as plsc

N_NODES = 10000
N_EDGES = 320000
D_IN = 128
D_HID = 256

NPAD = 10240
DUMP = 10000
NC = 2
NS = 16
CHUNK = 64
NCHUNK = 160
PART = 40
EPAD = NC * NS * NCHUNK * CHUNK
EPW = EPAD // (NC * NS)
ROWS_PER_TILE = NPAD // NS


def _sc_counts_body(dst_hbm, out_hbm, dst_v, cnt_v):
    c = lax.axis_index("c")
    s = lax.axis_index("s")
    w = c * NS + s

    def zero_body(i, _):
        cnt_v[i, pl.ds(0, 16)] = jnp.zeros((16,), jnp.float32)
        cnt_v[i, pl.ds(16, 16)] = jnp.zeros((16,), jnp.float32)
        cnt_v[i, pl.ds(32, 16)] = jnp.zeros((16,), jnp.float32)
        cnt_v[i, pl.ds(48, 16)] = jnp.zeros((16,), jnp.float32)
        cnt_v[i, pl.ds(64, 16)] = jnp.zeros((16,), jnp.float32)
        cnt_v[i, pl.ds(80, 16)] = jnp.zeros((16,), jnp.float32)
        cnt_v[i, pl.ds(96, 16)] = jnp.zeros((16,), jnp.float32)
        cnt_v[i, pl.ds(112, 16)] = jnp.zeros((16,), jnp.float32)
        return 0

    lax.fori_loop(0, NPAD // 128, zero_body, 0)
    pltpu.sync_copy(dst_hbm.at[pl.ds(w * EPW, EPW)], dst_v)
    ones = jnp.ones((16,), jnp.float32)

    def body(i, _):
        idx = dst_v[pl.ds(i * 16, 16)]
        row = lax.shift_right_logical(idx, 7)
        col = jnp.bitwise_and(idx, 127)
        plsc.addupdate_scatter(cnt_v, [row, col], ones)
        return 0

    lax.fori_loop(0, EPW // 16, body, 0)
    pltpu.sync_copy(cnt_v, out_hbm.at[w])


def _sc_scatter_body(table_hbm, src_hbm, dst_hbm, out_hbm, src_v, dst_v,
                     bufa, bufb, bufc, bufd, acc, sema, semb, semc, semd):
    c = lax.axis_index("c")
    s = lax.axis_index("s")
    w = c * NS + s

    def zero_body(i, _):
        bufa[i, pl.ds(0, 16)] = jnp.zeros((16,), jnp.float32)
        bufa[i, pl.ds(16, 16)] = jnp.zeros((16,), jnp.float32)
        bufa[i, pl.ds(32, 16)] = jnp.zeros((16,), jnp.float32)
        bufa[i, pl.ds(48, 16)] = jnp.zeros((16,), jnp.float32)
        bufa[i, pl.ds(64, 16)] = jnp.zeros((16,), jnp.float32)
        bufa[i, pl.ds(80, 16)] = jnp.zeros((16,), jnp.float32)
        bufa[i, pl.ds(96, 16)] = jnp.zeros((16,), jnp.float32)
        bufa[i, pl.ds(112, 16)] = jnp.zeros((16,), jnp.float32)
        return 0

    lax.fori_loop(0, CHUNK, zero_body, 0)
    for k in range(ROWS_PER_TILE // CHUNK):
        pltpu.sync_copy(bufa,
                        acc.at[pl.ds(s * ROWS_PER_TILE + k * CHUNK, CHUNK)])
    plsc.subcore_barrier()

    bufs = (bufa, bufb, bufc, bufd)
    sems = (sema, semb, semc, semd)

    def gather(e, b):
        pltpu.async_copy(table_hbm.at[src_v.at[e]], bufs[b], sems[b])

    def wait_scatter(e, b):
        pltpu.make_async_copy(table_hbm.at[src_v.at[e]], bufs[b],
                              sems[b]).wait()
        pltpu.sync_copy(bufs[b], acc.at[dst_v.at[e]], add=True)

    for part in range(NCHUNK // PART):
        base = w * NCHUNK + part * PART
        pltpu.sync_copy(src_hbm.at[pl.ds(base, PART)], src_v)
        pltpu.sync_copy(dst_hbm.at[pl.ds(base, PART)], dst_v)
        for b in range(4):
            gather(b, b)

        def body(t, _):
            e = 4 * t
            wait_scatter(e, 0)
            gather(e + 4, 0)
            wait_scatter(e + 1, 1)
            gather(e + 5, 1)
            wait_scatter(e + 2, 2)
            gather(e + 6, 2)
            wait_scatter(e + 3, 3)
            gather(e + 7, 3)
            return 0

        lax.fori_loop(0, PART // 4 - 1, body, 0)
        for b in range(4):
            wait_scatter(PART - 4 + b, b)

    plsc.subcore_barrier()
    pltpu.sync_copy(
        acc.at[pl.ds(s * ROWS_PER_TILE, ROWS_PER_TILE)],
        out_hbm.at[pl.ds(c * NPAD + s * ROWS_PER_TILE, ROWS_PER_TILE)])


@functools.cache
def _sc_kernels():
    mesh = plsc.VectorSubcoreMesh(core_axis_name="c", subcore_axis_name="s")
    params = pltpu.CompilerParams(needs_layout_passes=False)
    counts = pl.kernel(
        _sc_counts_body,
        out_type=jax.ShapeDtypeStruct((NC * NS, NPAD // 128, 128),
                                      jnp.float32),
        mesh=mesh,
        scratch_types=[
            pltpu.VMEM((EPW,), jnp.int32),
            pltpu.VMEM((NPAD // 128, 128), jnp.float32),
        ],
        compiler_params=params,
    )
    scatter = pl.kernel(
        _sc_scatter_body,
        out_type=jax.ShapeDtypeStruct((NC * NPAD, D_IN), jnp.float32),
        mesh=mesh,
        scratch_types=[
            pltpu.VMEM((PART, CHUNK), jnp.int32),
            pltpu.VMEM((PART, CHUNK), jnp.int32),
            pltpu.VMEM((CHUNK, D_IN), jnp.float32),
            pltpu.VMEM((CHUNK, D_IN), jnp.float32),
            pltpu.VMEM((CHUNK, D_IN), jnp.float32),
            pltpu.VMEM((CHUNK, D_IN), jnp.float32),
            pltpu.VMEM_SHARED((NPAD, D_IN), jnp.float32),
            pltpu.SemaphoreType.DMA,
            pltpu.SemaphoreType.DMA,
            pltpu.SemaphoreType.DMA,
            pltpu.SemaphoreType.DMA,
        ],
        compiler_params=params,
    )
    return counts, scatter


def _tc_dinv_body(cnt_ref, out_ref):
    deg = jnp.sum(cnt_ref[...], axis=0) + 1.0
    out_ref[...] = lax.rsqrt(deg)


def _tc_scale_body(x_ref, dinv_ref, out_ref):
    out_ref[...] = x_ref[...] * dinv_ref[...]


def _tc_mid_body(acc_ref, xs_ref, dinv_ref, w1_ref, b1_ref, w2_ref, out_ref):
    dinv = dinv_ref[...]
    z = (acc_ref[0] + acc_ref[1] + xs_ref[...]) * dinv
    h1 = jnp.dot(z, w1_ref[...], preferred_element_type=jnp.float32)
    h1 = jnp.maximum(h1 + b1_ref[...], 0.0)
    out_ref[...] = jnp.dot(h1, w2_ref[...],
                           preferred_element_type=jnp.float32) * dinv


def _tc_out_body(acc_ref, g2_ref, dinv_ref, b2_ref, out_ref):
    pre = (acc_ref[0] + acc_ref[1] + g2_ref[...]) * dinv_ref[...]
    out_ref[...] = jnp.maximum(pre + b2_ref[...], 0.0)


def kernel(x, edge_index, W1, b1, W2, b2):
    src = edge_index[0]
    dst = edge_index[1]
    pad = DUMP + jnp.arange(EPAD - N_EDGES, dtype=jnp.int32) % (NPAD - DUMP)
    src_p = jnp.concatenate([src, pad])
    dst_p = jnp.concatenate([dst, pad])
    src_r = src_p.reshape(NC * NS * NCHUNK, CHUNK)
    dst_r = dst_p.reshape(NC * NS * NCHUNK, CHUNK)

    _sc_counts, _sc_scatter = _sc_kernels()
    counts = _sc_counts(dst_p)

    dinv = pl.pallas_call(
        _tc_dinv_body,
        out_shape=jax.ShapeDtypeStruct((NPAD // 128, 128), jnp.float32),
    )(counts)
    dinv_col = dinv.reshape(NPAD, 1)

    xs = pl.pallas_call(
        _tc_scale_body,
        grid=(5,),
        in_specs=[
            pl.BlockSpec((2048, D_IN), lambda i: (i, 0)),
            pl.BlockSpec((2048, 1), lambda i: (i, 0)),
        ],
        out_specs=pl.BlockSpec((2048, D_IN), lambda i: (i, 0)),
        out_shape=jax.ShapeDtypeStruct((NPAD, D_IN), jnp.float32),
    )(x, dinv_col)

    acc1 = _sc_scatter(xs, src_r, dst_r).reshape(NC, NPAD, D_IN)

    g2 = pl.pallas_call(
        _tc_mid_body,
        grid=(8,),
        in_specs=[
            pl.BlockSpec((NC, 1280, D_IN), lambda i: (0, i, 0)),
            pl.BlockSpec((1280, D_IN), lambda i: (i, 0)),
            pl.BlockSpec((1280, 1), lambda i: (i, 0)),
            pl.BlockSpec((D_IN, D_HID), lambda i: (0, 0)),
            pl.BlockSpec((1, D_HID), lambda i: (0, 0)),
            pl.BlockSpec((D_HID, D_IN), lambda i: (0, 0)),
        ],
        out_specs=pl.BlockSpec((1280, D_IN), lambda i: (i, 0)),
        out_shape=jax.ShapeDtypeStruct((NPAD, D_IN), jnp.float32),
    )(acc1, xs, dinv_col, W1, b1.reshape(1, D_HID), W2)

    acc2 = _sc_scatter(g2, src_r, dst_r).reshape(NC, NPAD, D_IN)

    out = pl.pallas_call(
        _tc_out_body,
        grid=(5,),
        in_specs=[
            pl.BlockSpec((NC, 2000, D_IN), lambda i: (0, i, 0)),
            pl.BlockSpec((2000, D_IN), lambda i: (i, 0)),
            pl.BlockSpec((2000, 1), lambda i: (i, 0)),
            pl.BlockSpec((1, D_IN), lambda i: (0, 0)),
        ],
        out_specs=pl.BlockSpec((2000, D_IN), lambda i: (i, 0)),
        out_shape=jax.ShapeDtypeStruct((N_NODES, D_IN), jnp.float32),
    )(acc2, g2, dinv_col, b2.reshape(1, D_IN))

    return out


# speedup vs baseline: 39.9811x; 1.0089x over previous
import functools

import jax
import jax.numpy as jnp
from jax import lax
from jax.experimental import pallas as pl
from jax.experimental.pallas import tpu as pltpu
from jax.experimental.pallas import tpu_sc as plsc

N_NODES = 10000
N_EDGES = 320000
D_IN = 128
D_HID = 256

NPAD = 10240
DUMP = 10000
NC = 2
NS = 16
CHUNK = 64
NCHUNK = 160
PART = 40
EPAD = NC * NS * NCHUNK * CHUNK
EPW = EPAD // (NC * NS)
ROWS_PER_TILE = NPAD // NS


def _sc_counts_body(dst_hbm, out_hbm, dst_v, cnt_v):
    c = lax.axis_index("c")
    s = lax.axis_index("s")
    w = c * NS + s

    def zero_body(i, _):
        cnt_v[i, pl.ds(0, 16)] = jnp.zeros((16,), jnp.float32)
        cnt_v[i, pl.ds(16, 16)] = jnp.zeros((16,), jnp.float32)
        cnt_v[i, pl.ds(32, 16)] = jnp.zeros((16,), jnp.float32)
        cnt_v[i, pl.ds(48, 16)] = jnp.zeros((16,), jnp.float32)
        cnt_v[i, pl.ds(64, 16)] = jnp.zeros((16,), jnp.float32)
        cnt_v[i, pl.ds(80, 16)] = jnp.zeros((16,), jnp.float32)
        cnt_v[i, pl.ds(96, 16)] = jnp.zeros((16,), jnp.float32)
        cnt_v[i, pl.ds(112, 16)] = jnp.zeros((16,), jnp.float32)
        return 0

    lax.fori_loop(0, NPAD // 128, zero_body, 0)
    pltpu.sync_copy(dst_hbm.at[pl.ds(w * EPW, EPW)], dst_v)
    ones = jnp.ones((16,), jnp.float32)

    def body(i, _):
        idx = dst_v[pl.ds(i * 16, 16)]
        row = lax.shift_right_logical(idx, 7)
        col = jnp.bitwise_and(idx, 127)
        plsc.addupdate_scatter(cnt_v, [row, col], ones)
        return 0

    lax.fori_loop(0, EPW // 16, body, 0)
    pltpu.sync_copy(cnt_v, out_hbm.at[w])


def _sc_scatter_body(table_hbm, src_hbm, dst_hbm, out_hbm, src_v, dst_v,
                     bufa, bufb, bufc, bufd, acc, sema, semb, semc, semd):
    c = lax.axis_index("c")
    s = lax.axis_index("s")
    w = c * NS + s

    def zero_body(i, _):
        bufa[i, pl.ds(0, 16)] = jnp.zeros((16,), jnp.float32)
        bufa[i, pl.ds(16, 16)] = jnp.zeros((16,), jnp.float32)
        bufa[i, pl.ds(32, 16)] = jnp.zeros((16,), jnp.float32)
        bufa[i, pl.ds(48, 16)] = jnp.zeros((16,), jnp.float32)
        bufa[i, pl.ds(64, 16)] = jnp.zeros((16,), jnp.float32)
        bufa[i, pl.ds(80, 16)] = jnp.zeros((16,), jnp.float32)
        bufa[i, pl.ds(96, 16)] = jnp.zeros((16,), jnp.float32)
        bufa[i, pl.ds(112, 16)] = jnp.zeros((16,), jnp.float32)
        return 0

    lax.fori_loop(0, CHUNK, zero_body, 0)
    for k in range(ROWS_PER_TILE // CHUNK):
        pltpu.sync_copy(bufa,
                        acc.at[pl.ds(s * ROWS_PER_TILE + k * CHUNK, CHUNK)])
    plsc.subcore_barrier()

    bufs = (bufa, bufb, bufc, bufd)
    sems = (sema, semb, semc, semd)

    def gather(e, b):
        pltpu.async_copy(table_hbm.at[src_v.at[e]], bufs[b], sems[b])

    def wait_scatter(e, b):
        pltpu.make_async_copy(table_hbm.at[src_v.at[e]], bufs[b],
                              sems[b]).wait()
        pltpu.sync_copy(bufs[b], acc.at[dst_v.at[e]], add=True)

    for part in range(NCHUNK // PART):
        base = w * NCHUNK + part * PART
        pltpu.sync_copy(src_hbm.at[pl.ds(base, PART)], src_v)
        pltpu.sync_copy(dst_hbm.at[pl.ds(base, PART)], dst_v)
        for b in range(4):
            gather(b, b)

        def body(t, _):
            e = 4 * t
            wait_scatter(e, 0)
            gather(e + 4, 0)
            wait_scatter(e + 1, 1)
            gather(e + 5, 1)
            wait_scatter(e + 2, 2)
            gather(e + 6, 2)
            wait_scatter(e + 3, 3)
            gather(e + 7, 3)
            return 0

        lax.fori_loop(0, PART // 4 - 1, body, 0)
        for b in range(4):
            wait_scatter(PART - 4 + b, b)

    plsc.subcore_barrier()
    pltpu.sync_copy(
        acc.at[pl.ds(s * ROWS_PER_TILE, ROWS_PER_TILE)],
        out_hbm.at[pl.ds(c * NPAD + s * ROWS_PER_TILE, ROWS_PER_TILE)])


@functools.cache
def _sc_kernels():
    mesh = plsc.VectorSubcoreMesh(core_axis_name="c", subcore_axis_name="s")
    params = pltpu.CompilerParams(needs_layout_passes=False)
    counts = pl.kernel(
        _sc_counts_body,
        out_type=jax.ShapeDtypeStruct((NC * NS, NPAD // 128, 128),
                                      jnp.float32),
        mesh=mesh,
        scratch_types=[
            pltpu.VMEM((EPW,), jnp.int32),
            pltpu.VMEM((NPAD // 128, 128), jnp.float32),
        ],
        compiler_params=params,
    )
    scatter = pl.kernel(
        _sc_scatter_body,
        out_type=jax.ShapeDtypeStruct((NC * NPAD, D_IN), jnp.float32),
        mesh=mesh,
        scratch_types=[
            pltpu.VMEM((PART, CHUNK), jnp.int32),
            pltpu.VMEM((PART, CHUNK), jnp.int32),
            pltpu.VMEM((CHUNK, D_IN), jnp.float32),
            pltpu.VMEM((CHUNK, D_IN), jnp.float32),
            pltpu.VMEM((CHUNK, D_IN), jnp.float32),
            pltpu.VMEM((CHUNK, D_IN), jnp.float32),
            pltpu.VMEM_SHARED((NPAD, D_IN), jnp.float32),
            pltpu.SemaphoreType.DMA,
            pltpu.SemaphoreType.DMA,
            pltpu.SemaphoreType.DMA,
            pltpu.SemaphoreType.DMA,
        ],
        compiler_params=params,
    )
    return counts, scatter


def _tc_prep_body(cnt_ref, x_ref, dinv_ref, xs_ref):
    deg = jnp.sum(cnt_ref[...], axis=0) + 1.0
    dinv = lax.rsqrt(deg)
    eye = jnp.eye(128, dtype=jnp.float32)
    cols = [
        lax.dot_general(eye, dinv[r:r + 1], (((1,), (1,)), ((), ())),
                        preferred_element_type=jnp.float32)
        for r in range(8)
    ]
    dcol = jnp.concatenate(cols, axis=0)
    dinv_ref[...] = dcol
    xs_ref[...] = x_ref[...] * dcol


def _tc_mid_body(acc_ref, xs_ref, dinv_ref, w1_ref, b1_ref, w2_ref, out_ref):
    dinv = dinv_ref[...]
    z = (acc_ref[0] + acc_ref[1] + xs_ref[...]) * dinv
    h1 = jnp.dot(z, w1_ref[...], preferred_element_type=jnp.float32)
    h1 = jnp.maximum(h1 + b1_ref[...], 0.0)
    out_ref[...] = jnp.dot(h1, w2_ref[...],
                           preferred_element_type=jnp.float32) * dinv


def _tc_out_body(acc_ref, g2_ref, dinv_ref, b2_ref, out_ref):
    pre = (acc_ref[0] + acc_ref[1] + g2_ref[...]) * dinv_ref[...]
    out_ref[...] = jnp.maximum(pre + b2_ref[...], 0.0)


def kernel(x, edge_index, W1, b1, W2, b2):
    src = edge_index[0]
    dst = edge_index[1]
    pad = DUMP + jnp.arange(EPAD - N_EDGES, dtype=jnp.int32) % (NPAD - DUMP)
    src_p = jnp.concatenate([src, pad])
    dst_p = jnp.concatenate([dst, pad])
    src_r = src_p.reshape(NC * NS * NCHUNK, CHUNK)
    dst_r = dst_p.reshape(NC * NS * NCHUNK, CHUNK)

    _sc_counts, _sc_scatter = _sc_kernels()
    counts = _sc_counts(dst_p)

    dinv_col, xs = pl.pallas_call(
        _tc_prep_body,
        grid=(10,),
        in_specs=[
            pl.BlockSpec((NC * NS, 8, 128), lambda i: (0, i, 0)),
            pl.BlockSpec((1024, D_IN), lambda i: (i, 0)),
        ],
        out_specs=[
            pl.BlockSpec((1024, 1), lambda i: (i, 0)),
            pl.BlockSpec((1024, D_IN), lambda i: (i, 0)),
        ],
        out_shape=[
            jax.ShapeDtypeStruct((NPAD, 1), jnp.float32),
            jax.ShapeDtypeStruct((NPAD, D_IN), jnp.float32),
        ],
    )(counts, x)

    acc1 = _sc_scatter(xs, src_r, dst_r).reshape(NC, NPAD, D_IN)

    g2 = pl.pallas_call(
        _tc_mid_body,
        grid=(8,),
        in_specs=[
            pl.BlockSpec((NC, 1280, D_IN), lambda i: (0, i, 0)),
            pl.BlockSpec((1280, D_IN), lambda i: (i, 0)),
            pl.BlockSpec((1280, 1), lambda i: (i, 0)),
            pl.BlockSpec((D_IN, D_HID), lambda i: (0, 0)),
            pl.BlockSpec((1, D_HID), lambda i: (0, 0)),
            pl.BlockSpec((D_HID, D_IN), lambda i: (0, 0)),
        ],
        out_specs=pl.BlockSpec((1280, D_IN), lambda i: (i, 0)),
        out_shape=jax.ShapeDtypeStruct((NPAD, D_IN), jnp.float32),
    )(acc1, xs, dinv_col, W1, b1.reshape(1, D_HID), W2)

    acc2 = _sc_scatter(g2, src_r, dst_r).reshape(NC, NPAD, D_IN)

    out = pl.pallas_call(
        _tc_out_body,
        grid=(5,),
        in_specs=[
            pl.BlockSpec((NC, 2000, D_IN), lambda i: (0, i, 0)),
            pl.BlockSpec((2000, D_IN), lambda i: (i, 0)),
            pl.BlockSpec((2000, 1), lambda i: (i, 0)),
            pl.BlockSpec((1, D_IN), lambda i: (0, 0)),
        ],
        out_specs=pl.BlockSpec((2000, D_IN), lambda i: (i, 0)),
        out_shape=jax.ShapeDtypeStruct((N_NODES, D_IN), jnp.float32),
    )(acc2, g2, dinv_col, b2.reshape(1, D_IN))

    return out
